# baseline (speedup 1.0000x reference)
import functools

import jax
import jax.numpy as jnp
from jax import lax
from jax.experimental import pallas as pl
from jax.experimental.pallas import tpu as pltpu
from jax.experimental.pallas import tpu_sc as plsc

N = 10000
D = 128
E = 320000

NC = 2
NS = 16
NW = NC * NS
C = 128
R = 16
SUP = 5
EPAD = NW * SUP * R * C
NPAD = 10112
ROWS_PT = NPAD // NS

_mesh = plsc.VectorSubcoreMesh(
    core_axis_name="c", subcore_axis_name="s", num_cores=NC, num_subcores=NS)


DT = 80


def _deg_body(dst_hbm, zeros_hbm, out_hbm, dst_v, deg_v):
    cid = lax.axis_index("c")
    sid = lax.axis_index("s")
    wid = sid * NC + cid
    ones = jnp.ones((16,), jnp.float32)
    pltpu.sync_copy(zeros_hbm, deg_v)

    def sup_body(sp, carry):
        pltpu.sync_copy(dst_hbm.at[wid * SUP + sp], dst_v)

        def row_body(r, carry2):
            row = dst_v.at[r]
            for g in range(C // 16):
                idx = row[pl.ds(g * 16, 16)]
                plsc.addupdate_scatter(deg_v, [idx], ones)
            return carry2

        lax.fori_loop(0, R, row_body, 0)
        return carry

    lax.fori_loop(0, SUP, sup_body, 0)
    pltpu.sync_copy(deg_v, out_hbm.at[pl.ds(wid * (DT * 128), DT * 128)])


_deg_call = pl.kernel(
    _deg_body,
    out_type=jax.ShapeDtypeStruct((NW * DT * 128,), jnp.float32),
    mesh=_mesh,
    scratch_types=[
        pltpu.VMEM((R, C), jnp.int32),
        pltpu.VMEM((DT * 128,), jnp.float32),
    ],
    compiler_params=pltpu.CompilerParams(needs_layout_passes=False),
)


C2 = 64
R2 = 32
TS = NW * SUP
N0 = 5
N1 = SUP * 2 - N0


def _conv_body(h_hbm, src_hbm, dst_hbm, out_hbm,
               src_v, dst_v, b0, b1, b2, b3, acc_sh,
               g0, g1, g2, g3, s0, s1, s2, s3):
    bufs = [b0, b1, b2, b3]
    gsems = [g0, g1, g2, g3]
    ssems = [s0, s1, s2, s3]
    cid = lax.axis_index("c")
    sid = lax.axis_index("s")
    with jax.named_scope("acc_init"):
        pltpu.sync_copy(h_hbm.at[pl.ds(sid * ROWS_PT, ROWS_PT)],
                        acc_sh.at[pl.ds(sid * ROWS_PT, ROWS_PT)])
        plsc.subcore_barrier()
    base = jnp.where(cid == 0, sid * N0, NS * N0 + sid * N1)
    nsup = jnp.where(cid == 0, N0, N1)
    NQ = R2 // 4

    def g_issue(j, b):
        pltpu.async_copy(h_hbm.at[src_v.at[j]], bufs[b], gsems[b])

    def g_wait(j, b):
        pltpu.make_async_copy(h_hbm.at[src_v.at[j]], bufs[b], gsems[b]).wait()

    def s_issue(j, b):
        pltpu.async_copy(bufs[b], acc_sh.at[dst_v.at[j]], ssems[b], add=True)

    def s_wait(j, b):
        pltpu.make_async_copy(bufs[b], acc_sh.at[dst_v.at[j]], ssems[b]).wait()

    def sup_body(sp, carry):
        pltpu.sync_copy(src_hbm.at[base + sp], src_v)
        pltpu.sync_copy(dst_hbm.at[base + sp], dst_v)
        g_issue(0, 0)

        def quad(q, carry2):
            j0 = 4 * q
            for k in range(4):
                j = j0 + k
                if k < 3:
                    g_issue(j + 1, (k + 1) % 4)
                else:
                    @pl.when(q < NQ - 1)
                    def _():
                        g_issue(j + 1, 0)
                g_wait(j, k)
                if k >= 2:
                    s_wait(j - 2, k - 2)
                else:
                    @pl.when(q >= 1)
                    def _():
                        s_wait(j - 2, (k - 2) % 4)
                s_issue(j, k)
            return carry2

        lax.fori_loop(0, NQ, quad, 0)
        s_wait(R2 - 2, (R2 - 2) % 4)
        s_wait(R2 - 1, (R2 - 1) % 4)
        return carry

    with jax.named_scope("edges"):
        lax.fori_loop(0, nsup, sup_body, 0)
        plsc.subcore_barrier()
    with jax.named_scope("writeback"):
        pltpu.sync_copy(acc_sh.at[pl.ds(sid * ROWS_PT, ROWS_PT)],
                        out_hbm.at[cid, pl.ds(sid * ROWS_PT, ROWS_PT)])


_conv_call = pl.kernel(
    _conv_body,
    out_type=jax.ShapeDtypeStruct((NC, NPAD, D), jnp.float32),
    mesh=_mesh,
    scratch_types=[
        pltpu.VMEM((R2, C2), jnp.int32),
        pltpu.VMEM((R2, C2), jnp.int32),
        pltpu.VMEM((C2, D), jnp.float32),
        pltpu.VMEM((C2, D), jnp.float32),
        pltpu.VMEM((C2, D), jnp.float32),
        pltpu.VMEM((C2, D), jnp.float32),
        pltpu.VMEM_SHARED((NPAD, D), jnp.float32),
        pltpu.SemaphoreType.DMA,
        pltpu.SemaphoreType.DMA,
        pltpu.SemaphoreType.DMA,
        pltpu.SemaphoreType.DMA,
        pltpu.SemaphoreType.DMA,
        pltpu.SemaphoreType.DMA,
        pltpu.SemaphoreType.DMA,
        pltpu.SemaphoreType.DMA,
    ],
)


def _tc_deg_body(degp_ref, out_ref):
    out_ref[...] = lax.rsqrt(jnp.sum(degp_ref[...], axis=0) + 1.0)


_tc_deg = pl.pallas_call(
    _tc_deg_body, out_shape=jax.ShapeDtypeStruct((DT, 128), jnp.float32))


def _tc_pre_body(x_ref, w_ref, dinv_ref, out_ref):
    dinv = dinv_ref[...]
    h = jnp.dot(x_ref[...], w_ref[...], preferred_element_type=jnp.float32)
    out_ref[0:N, :] = h * dinv
    out_ref[N:NPAD, :] = jnp.zeros((NPAD - N, D), jnp.float32)


_tc_pre = pl.pallas_call(
    _tc_pre_body, out_shape=jax.ShapeDtypeStruct((NPAD, D), jnp.float32))


def _tc_mid_body(acc_ref, hp_ref, dinv_ref, b_ref, g_ref, beta_ref, w_ref,
                 out_ref):
    dinv = dinv_ref[...]
    s = acc_ref[0, 0:N, :] + acc_ref[1, 0:N, :] - hp_ref[0:N, :]
    z = s * dinv + b_ref[...]
    mean = jnp.mean(z, axis=0, keepdims=True)
    var = jnp.mean((z - mean) ** 2, axis=0, keepdims=True)
    zh = (z - mean) * lax.rsqrt(var + 1e-5) * g_ref[...] + beta_ref[...]
    r = jnp.maximum(zh, 0.0)
    out_ref[0:N, :] = jnp.dot(
        r, w_ref[...], preferred_element_type=jnp.float32) * dinv
    out_ref[N:NPAD, :] = jnp.zeros((NPAD - N, D), jnp.float32)


_tc_mid = pl.pallas_call(
    _tc_mid_body, out_shape=jax.ShapeDtypeStruct((NPAD, D), jnp.float32))


def _tc_post_body(acc_ref, hp_ref, dinv_ref, b_ref, out_ref):
    dinv = dinv_ref[...]
    s = acc_ref[0, 0:N, :] + acc_ref[1, 0:N, :] - hp_ref[0:N, :]
    out_ref[...] = s * dinv + b_ref[...]


_tc_post = pl.pallas_call(
    _tc_post_body, out_shape=jax.ShapeDtypeStruct((N, D), jnp.float32))


def kernel(x, edge_index, W1, b1, g1, beta1, W2, b2, g2, beta2, W3, b3):
    src = edge_index[0]
    dst = edge_index[1]
    pad = EPAD - E
    pad_src = (jnp.arange(pad, dtype=jnp.int32) * 131) % N
    pad_dst = N + (jnp.arange(pad, dtype=jnp.int32) % (NPAD - N))
    src_flat = jnp.concatenate([src, pad_src])
    dst_flat = jnp.concatenate([dst, pad_dst])
    src4d = src_flat.reshape(TS, R2, C2)
    dst4d = dst_flat.reshape(TS, R2, C2)
    dst3d = dst_flat.reshape(TS, R, C)
    zeros_tab = jnp.zeros((DT * 128,), jnp.float32)

    degp = _deg_call(dst3d, zeros_tab).reshape(NW, DT, 128)
    dinv_tab = _tc_deg(degp)
    dinv_col = dinv_tab.reshape(DT * 128)[:N].reshape(N, 1)

    b1r, g1r, beta1r = b1.reshape(1, D), g1.reshape(1, D), beta1.reshape(1, D)
    b2r, g2r, beta2r = b2.reshape(1, D), g2.reshape(1, D), beta2.reshape(1, D)
    b3r = b3.reshape(1, D)

    h1p = _tc_pre(x, W1, dinv_col)
    acc1 = _conv_call(h1p, src4d, dst4d)
    h2p = _tc_mid(acc1, h1p, dinv_col, b1r, g1r, beta1r, W2)
    acc2 = _conv_call(h2p, src4d, dst4d)
    h3p = _tc_mid(acc2, h2p, dinv_col, b2r, g2r, beta2r, W3)
    acc3 = _conv_call(h3p, src4d, dst4d)
    q = _tc_post(acc3, h3p, dinv_col, b3r)
    return q


# speedup vs baseline: 1.0800x; 1.0800x over previous
import functools

import jax
import jax.numpy as jnp
from jax import lax
from jax.experimental import pallas as pl
from jax.experimental.pallas import tpu as pltpu
from jax.experimental.pallas import tpu_sc as plsc

N = 10000
D = 128
E = 320000

NC = 2
NS = 16
NW = NC * NS
C = 128
R = 16
SUP = 5
EPAD = NW * SUP * R * C
NPAD = 10112
ROWS_PT = NPAD // NS

_mesh = plsc.VectorSubcoreMesh(
    core_axis_name="c", subcore_axis_name="s", num_cores=NC, num_subcores=NS)


DT = 80


def _deg_body(dst_hbm, zeros_hbm, out_hbm, dst_v, deg_v):
    cid = lax.axis_index("c")
    sid = lax.axis_index("s")
    wid = sid * NC + cid
    ones = jnp.ones((16,), jnp.float32)
    pltpu.sync_copy(zeros_hbm, deg_v)

    def sup_body(sp, carry):
        pltpu.sync_copy(dst_hbm.at[wid * SUP + sp], dst_v)

        def row_body(r, carry2):
            row = dst_v.at[r]
            for g in range(C // 16):
                idx = row[pl.ds(g * 16, 16)]
                plsc.addupdate_scatter(deg_v, [idx], ones)
            return carry2

        lax.fori_loop(0, R, row_body, 0)
        return carry

    lax.fori_loop(0, SUP, sup_body, 0)
    pltpu.sync_copy(deg_v, out_hbm.at[pl.ds(wid * (DT * 128), DT * 128)])


_deg_call = pl.kernel(
    _deg_body,
    out_type=jax.ShapeDtypeStruct((NW * DT * 128,), jnp.float32),
    mesh=_mesh,
    scratch_types=[
        pltpu.VMEM((R, C), jnp.int32),
        pltpu.VMEM((DT * 128,), jnp.float32),
    ],
    compiler_params=pltpu.CompilerParams(needs_layout_passes=False),
)


C2 = 64
R2 = 32
TS = NW * SUP
N0 = 5
N1 = SUP * 2 - N0


def _conv_body(h_hbm, src_hbm, dst_hbm, out_hbm,
               src_v, dst_v, buf0, buf1, acc_sh, g0, g1):
    cid = lax.axis_index("c")
    sid = lax.axis_index("s")
    with jax.named_scope("acc_init"):
        pltpu.sync_copy(h_hbm.at[pl.ds(sid * ROWS_PT, ROWS_PT)],
                        acc_sh.at[pl.ds(sid * ROWS_PT, ROWS_PT)])
        plsc.subcore_barrier()
    base = jnp.where(cid == 0, sid * N0, NS * N0 + sid * N1)
    nsup = jnp.where(cid == 0, N0, N1)

    def sup_body(sp, carry):
        pltpu.sync_copy(src_hbm.at[base + sp], src_v)
        pltpu.sync_copy(dst_hbm.at[base + sp], dst_v)
        pltpu.async_copy(h_hbm.at[src_v.at[0]], buf0, g0)

        def pair(t, carry2):
            j0 = 2 * t
            pltpu.async_copy(h_hbm.at[src_v.at[j0 + 1]], buf1, g1)
            pltpu.make_async_copy(h_hbm.at[src_v.at[j0]], buf0, g0).wait()
            pltpu.sync_copy(buf0, acc_sh.at[dst_v.at[j0]], add=True)

            @pl.when(t < R // 2 - 1)
            def _():
                pltpu.async_copy(h_hbm.at[src_v.at[j0 + 2]], buf0, g0)

            pltpu.make_async_copy(h_hbm.at[src_v.at[j0 + 1]], buf1, g1).wait()
            pltpu.sync_copy(buf1, acc_sh.at[dst_v.at[j0 + 1]], add=True)
            return carry2

        lax.fori_loop(0, R // 2, pair, 0)
        return carry

    with jax.named_scope("edges"):
        lax.fori_loop(0, nsup, sup_body, 0)
        plsc.subcore_barrier()
    with jax.named_scope("writeback"):
        pltpu.sync_copy(acc_sh.at[pl.ds(sid * ROWS_PT, ROWS_PT)],
                        out_hbm.at[cid, pl.ds(sid * ROWS_PT, ROWS_PT)])


_conv_call = pl.kernel(
    _conv_body,
    out_type=jax.ShapeDtypeStruct((NC, NPAD, D), jnp.float32),
    mesh=_mesh,
    scratch_types=[
        pltpu.VMEM((R, C), jnp.int32),
        pltpu.VMEM((R, C), jnp.int32),
        pltpu.VMEM((C, D), jnp.float32),
        pltpu.VMEM((C, D), jnp.float32),
        pltpu.VMEM_SHARED((NPAD, D), jnp.float32),
        pltpu.SemaphoreType.DMA,
        pltpu.SemaphoreType.DMA,
    ],
)


def _tc_deg_body(degp_ref, out_ref):
    out_ref[...] = lax.rsqrt(jnp.sum(degp_ref[...], axis=0) + 1.0)


_tc_deg = pl.pallas_call(
    _tc_deg_body, out_shape=jax.ShapeDtypeStruct((DT, 128), jnp.float32))


def _tc_pre_body(x_ref, w_ref, dinv_ref, out_ref):
    dinv = dinv_ref[...]
    h = jnp.dot(x_ref[...], w_ref[...], preferred_element_type=jnp.float32)
    out_ref[0:N, :] = h * dinv
    out_ref[N:NPAD, :] = jnp.zeros((NPAD - N, D), jnp.float32)


_tc_pre = pl.pallas_call(
    _tc_pre_body, out_shape=jax.ShapeDtypeStruct((NPAD, D), jnp.float32))


def _tc_mid_body(acc_ref, hp_ref, dinv_ref, b_ref, g_ref, beta_ref, w_ref,
                 out_ref):
    dinv = dinv_ref[...]
    s = acc_ref[0, 0:N, :] + acc_ref[1, 0:N, :] - hp_ref[0:N, :]
    z = s * dinv + b_ref[...]
    mean = jnp.mean(z, axis=0, keepdims=True)
    var = jnp.mean((z - mean) ** 2, axis=0, keepdims=True)
    zh = (z - mean) * lax.rsqrt(var + 1e-5) * g_ref[...] + beta_ref[...]
    r = jnp.maximum(zh, 0.0)
    out_ref[0:N, :] = jnp.dot(
        r, w_ref[...], preferred_element_type=jnp.float32) * dinv
    out_ref[N:NPAD, :] = jnp.zeros((NPAD - N, D), jnp.float32)


_tc_mid = pl.pallas_call(
    _tc_mid_body, out_shape=jax.ShapeDtypeStruct((NPAD, D), jnp.float32))


def _tc_post_body(acc_ref, hp_ref, dinv_ref, b_ref, out_ref):
    dinv = dinv_ref[...]
    s = acc_ref[0, 0:N, :] + acc_ref[1, 0:N, :] - hp_ref[0:N, :]
    out_ref[...] = s * dinv + b_ref[...]


_tc_post = pl.pallas_call(
    _tc_post_body, out_shape=jax.ShapeDtypeStruct((N, D), jnp.float32))


def kernel(x, edge_index, W1, b1, g1, beta1, W2, b2, g2, beta2, W3, b3):
    src = edge_index[0]
    dst = edge_index[1]
    pad = EPAD - E
    pad_src = (jnp.arange(pad, dtype=jnp.int32) * 131) % N
    pad_dst = N + (jnp.arange(pad, dtype=jnp.int32) % (NPAD - N))
    src_flat = jnp.concatenate([src, pad_src])
    dst_flat = jnp.concatenate([dst, pad_dst])
    src4d = src_flat.reshape(TS, R, C)
    dst4d = dst_flat.reshape(TS, R, C)
    zeros_tab = jnp.zeros((DT * 128,), jnp.float32)

    degp = _deg_call(dst4d, zeros_tab).reshape(NW, DT, 128)
    dinv_tab = _tc_deg(degp)
    dinv_col = dinv_tab.reshape(DT * 128)[:N].reshape(N, 1)

    b1r, g1r, beta1r = b1.reshape(1, D), g1.reshape(1, D), beta1.reshape(1, D)
    b2r, g2r, beta2r = b2.reshape(1, D), g2.reshape(1, D), beta2.reshape(1, D)
    b3r = b3.reshape(1, D)

    h1p = _tc_pre(x, W1, dinv_col)
    acc1 = _conv_call(h1p, src4d, dst4d)
    h2p = _tc_mid(acc1, h1p, dinv_col, b1r, g1r, beta1r, W2)
    acc2 = _conv_call(h2p, src4d, dst4d)
    h3p = _tc_mid(acc2, h2p, dinv_col, b2r, g2r, beta2r, W3)
    acc3 = _conv_call(h3p, src4d, dst4d)
    q = _tc_post(acc3, h3p, dinv_col, b3r)
    return q
